# baseline (speedup 1.0000x reference)
import functools

import jax
import jax.numpy as jnp
from jax import lax
from jax.experimental import pallas as pl
from jax.experimental.pallas import tpu as pltpu
from jax.experimental.pallas import tpu_sc as plsc

_BASE_HU = -2.0
_BASE_NORM = 0.0

_NW = 32
_CH = 8192
_UNROLL = 4


def _cumsum16(x, tmp_v, iot):
    acc = x
    for s in (1, 2, 4, 8):
        tmp_v[...] = acc
        t = plsc.load_gather(tmp_v, [jnp.maximum(iot - s, 0)])
        acc = acc + jnp.where(iot >= s, t, 0.0)
    return acc


def _build_tables(hu_v, norm_v, bt_v, a_v, c_v, tmp_v):
    iot = lax.iota(jnp.int32, 16)
    inf = jnp.float32(jnp.inf)
    for j in range(4):
        habs = jnp.abs(hu_v[j])
        nabs = jnp.abs(norm_v[j])
        H = _cumsum16(habs, tmp_v, iot)
        N = _cumsum16(nabs, tmp_v, iot)
        Hprev = H - habs
        Nprev = N - nabs
        k = nabs / habs
        mid = (iot >= 1) & (iot <= 7)
        avec = jnp.where(mid, k, 0.0)
        cin = Nprev - k * Hprev
        cvec = jnp.where(iot == 8, Nprev + _BASE_NORM, jnp.where(mid, cin, 0.0))
        bvec = jnp.where(iot <= 7, _BASE_HU + H, inf)
        bt_v[j + 1] = bvec
        a_v[j + 1] = avec
        c_v[j + 1] = cvec


def _splats(bt_v, a_v, c_v, j):
    jv = jnp.full((16,), j + 1, jnp.int32)
    bs = [plsc.load_gather(bt_v, [jv, jnp.full((16,), i, jnp.int32)]) for i in range(8)]
    as_ = [plsc.load_gather(a_v, [jv, jnp.full((16,), i, jnp.int32)]) for i in range(1, 8)]
    cs = [plsc.load_gather(c_v, [jv, jnp.full((16,), i, jnp.int32)]) for i in range(1, 8)]
    top = plsc.load_gather(c_v, [jv, jnp.full((16,), 8, jnp.int32)])
    return bs, as_, cs, top


def _make_body(nch):
    def _sc_body(img_hbm, hu_hbm, norm_hbm, out_hbm,
                 hu_v, norm_v, bt_v, a_v, c_v, tmp_v, x_v, y_v,
                 in_sems, out_sems):
        c = lax.axis_index("c")
        s = lax.axis_index("s")
        wid = s * 2 + c
        b = wid // 16
        col0 = (wid % 16) * (_CH * nch)

        pltpu.sync_copy(hu_hbm, hu_v)
        pltpu.sync_copy(norm_hbm, norm_v)
        _build_tables(hu_v, norm_v, bt_v, a_v, c_v, tmp_v)

        def in_copy(g, buf):
            return pltpu.make_async_copy(
                img_hbm.at[b, pl.ds(col0 + g * _CH, _CH)], x_v.at[buf], in_sems.at[buf])

        def out_copy(g, buf, j):
            return pltpu.make_async_copy(
                y_v.at[buf, j], out_hbm.at[b * 4 + j, pl.ds(col0 + g * _CH, _CH)],
                out_sems.at[buf])

        in_copy(0, 0).start()

        def chunk_body(g, _):
            buf = lax.rem(g, 2)
            in_copy(g, buf).wait()

            @pl.when(g + 1 < nch)
            def _():
                in_copy(g + 1, lax.rem(g + 1, 2)).start()

            @pl.when(g >= 2)
            def _():
                for j in range(4):
                    out_copy(g - 2, buf, j).wait()

            for j in range(4):
                bs, as_, cs, top = _splats(bt_v, a_v, c_v, j)

                def vec_body(i, _, buf=buf, j=j, bs=bs, as_=as_, cs=cs, top=top):
                    for u in range(_UNROLL):
                        off = (i * _UNROLL + u) * 16
                        x = x_v[buf, pl.ds(off, 16)]
                        y = jnp.zeros((16,), jnp.float32)
                        for t in range(7):
                            y = jnp.where(x >= bs[t], as_[t] * x + cs[t], y)
                        y = jnp.where(x >= bs[7], top, y)
                        y_v[buf, j, pl.ds(off, 16)] = y
                    return 0

                lax.fori_loop(0, _CH // (16 * _UNROLL), vec_body, 0)
                out_copy(g, buf, j).start()
            return 0

        lax.fori_loop(0, nch, chunk_body, 0)
        for g in (nch - 2, nch - 1):
            for j in range(4):
                out_copy(g, g % 2, j).wait()

    return _sc_body


def _sc_part(xa, hu_lis, norm_lis, B, Ma):
    pw = B * Ma // _NW
    nch = pw // _CH
    assert pw % _CH == 0
    hu16 = jnp.pad(hu_lis, ((0, 0), (0, 8)))
    norm16 = jnp.pad(norm_lis, ((0, 0), (0, 8)))
    mesh = plsc.VectorSubcoreMesh(core_axis_name="c", subcore_axis_name="s")
    f = pl.kernel(
        _make_body(nch),
        out_type=jax.ShapeDtypeStruct((B * 4, Ma), jnp.float32),
        mesh=mesh,
        scratch_types=[
            pltpu.VMEM((4, 16), jnp.float32),
            pltpu.VMEM((4, 16), jnp.float32),
            pltpu.VMEM((5, 16), jnp.float32),
            pltpu.VMEM((5, 16), jnp.float32),
            pltpu.VMEM((5, 16), jnp.float32),
            pltpu.VMEM((16,), jnp.float32),
            pltpu.VMEM((2, _CH), jnp.float32),
            pltpu.VMEM((2, 4, _CH), jnp.float32),
            pltpu.SemaphoreType.DMA((2,)),
            pltpu.SemaphoreType.DMA((2,)),
        ],
        compiler_params=pltpu.CompilerParams(needs_layout_passes=False),
    )
    out = f(xa, hu16, norm16)
    return out.reshape(B, 4, Ma)


_SUB = 512
_SC_BLOCKS = 4


def _tc_body(hu_ref, norm_ref, x_ref, out_ref):
    x = x_ref[0, 0]
    for j in range(4):
        h_low = jnp.abs(hu_ref[j, 0])
        n_low = jnp.abs(norm_ref[j, 0])
        y = jnp.zeros_like(x)
        for i in range(1, 8):
            h_high = h_low + jnp.abs(hu_ref[j, i])
            n_high = n_low + jnp.abs(norm_ref[j, i])
            k = (n_high - n_low) / (h_high - h_low)
            c = n_low - k * h_low
            y = jnp.where(x >= _BASE_HU + h_low, k * x + c, y)
            h_low, n_low = h_high, n_high
        y = jnp.where(x >= _BASE_HU + h_low, n_low + _BASE_NORM, y)
        out_ref[0, j, 0] = y


def _tc_part(xb, hu_lis, norm_lis, B, nblk):
    x = xb.reshape(B, nblk, _SUB, 256)
    out = pl.pallas_call(
        _tc_body,
        grid=(B, nblk),
        in_specs=[
            pl.BlockSpec(memory_space=pltpu.SMEM),
            pl.BlockSpec(memory_space=pltpu.SMEM),
            pl.BlockSpec((1, 1, _SUB, 256), lambda b, i: (b, i, 0, 0)),
        ],
        out_specs=pl.BlockSpec((1, 4, 1, _SUB, 256), lambda b, i: (b, 0, i, 0, 0)),
        out_shape=jax.ShapeDtypeStruct((B, 4, nblk, _SUB, 256), jnp.float32),
    )(hu_lis, norm_lis, x)
    return out.reshape(B, 4, nblk * _SUB * 256)


def kernel(img, hu_lis, norm_lis):
    B, C, D, H, W = img.shape
    M = D * H * W
    x = img.reshape(B, M)
    Ma = _SC_BLOCKS * _SUB * 256
    sc = _sc_part(x[:, :Ma], hu_lis, norm_lis, B, Ma)
    tc = _tc_part(x[:, Ma:], hu_lis, norm_lis, B, 16 - _SC_BLOCKS)
    out = jnp.concatenate([sc, tc], axis=2)
    return out.reshape(B, 4, D, H, W)


# speedup vs baseline: 2.8477x; 2.8477x over previous
import functools

import jax
import jax.numpy as jnp
from jax import lax
from jax.experimental import pallas as pl
from jax.experimental.pallas import tpu as pltpu
from jax.experimental.pallas import tpu_sc as plsc

_BASE_HU = -2.0
_BASE_NORM = 0.0

_NW = 32
_CH = 8192
_UNROLL = 4
_SUB = 512
_SC_BLOCKS = 3


def _cumsum16(x, tmp_v, iot):
    acc = x
    for s in (1, 2, 4, 8):
        tmp_v[...] = acc
        t = plsc.load_gather(tmp_v, [jnp.maximum(iot - s, 0)])
        acc = acc + jnp.where(iot >= s, t, 0.0)
    return acc


def _build_tables(hu_v, norm_v, bt_v, a_v, c_v, tmp_v):
    iot = lax.iota(jnp.int32, 16)
    inf = jnp.float32(jnp.inf)
    for j in range(4):
        habs = jnp.abs(hu_v[j])
        nabs = jnp.abs(norm_v[j])
        H = _cumsum16(habs, tmp_v, iot)
        N = _cumsum16(nabs, tmp_v, iot)
        Hprev = H - habs
        Nprev = N - nabs
        k = nabs / habs
        mid = (iot >= 1) & (iot <= 7)
        avec = jnp.where(mid, k, 0.0)
        cin = Nprev - k * Hprev
        cvec = jnp.where(iot == 8, Nprev + _BASE_NORM, jnp.where(mid, cin, 0.0))
        bvec = jnp.where(iot <= 7, _BASE_HU + H, inf)
        bt_v[j + 1] = bvec
        a_v[j + 1] = avec
        c_v[j + 1] = cvec


def _splats(bt_v, a_v, c_v, j):
    jv = jnp.full((16,), j + 1, jnp.int32)
    bs = [plsc.load_gather(bt_v, [jv, jnp.full((16,), i, jnp.int32)]) for i in range(8)]
    as_ = [plsc.load_gather(a_v, [jv, jnp.full((16,), i, jnp.int32)]) for i in range(1, 8)]
    cs = [plsc.load_gather(c_v, [jv, jnp.full((16,), i, jnp.int32)]) for i in range(1, 8)]
    top = plsc.load_gather(c_v, [jv, jnp.full((16,), 8, jnp.int32)])
    return bs, as_, cs, top


def _make_body(nch):
    def _sc_body(img_hbm, hu_hbm, norm_hbm, out_hbm,
                 hu_v, norm_v, bt_v, a_v, c_v, tmp_v, x_v, y_v,
                 in_sems, out_sems):
        c = lax.axis_index("c")
        s = lax.axis_index("s")
        wid = s * 2 + c
        b = wid // 16
        col0 = (wid % 16) * (_CH * nch)

        pltpu.sync_copy(hu_hbm, hu_v)
        pltpu.sync_copy(norm_hbm, norm_v)
        _build_tables(hu_v, norm_v, bt_v, a_v, c_v, tmp_v)

        def in_copy(g, buf):
            return pltpu.make_async_copy(
                img_hbm.at[b, pl.ds(col0 + g * _CH, _CH)], x_v.at[buf], in_sems.at[buf])

        def out_copy(g, buf, j):
            return pltpu.make_async_copy(
                y_v.at[buf, j], out_hbm.at[b * 4 + j, pl.ds(col0 + g * _CH, _CH)],
                out_sems.at[buf])

        in_copy(0, 0).start()

        def chunk_body(g, _):
            buf = lax.rem(g, 2)
            in_copy(g, buf).wait()

            @pl.when(g + 1 < nch)
            def _():
                in_copy(g + 1, lax.rem(g + 1, 2)).start()

            @pl.when(g >= 2)
            def _():
                for j in range(4):
                    out_copy(g - 2, buf, j).wait()

            for j in range(4):
                bs, as_, cs, top = _splats(bt_v, a_v, c_v, j)

                def vec_body(i, _, buf=buf, j=j, bs=bs, as_=as_, cs=cs, top=top):
                    for u in range(_UNROLL):
                        off = (i * _UNROLL + u) * 16
                        x = x_v[buf, pl.ds(off, 16)]
                        y = jnp.zeros((16,), jnp.float32)
                        for t in range(7):
                            y = jnp.where(x >= bs[t], as_[t] * x + cs[t], y)
                        y = jnp.where(x >= bs[7], top, y)
                        y_v[buf, j, pl.ds(off, 16)] = y
                    return 0

                lax.fori_loop(0, _CH // (16 * _UNROLL), vec_body, 0)
                out_copy(g, buf, j).start()
            return 0

        lax.fori_loop(0, nch, chunk_body, 0)
        for g in (nch - 2, nch - 1):
            for j in range(4):
                out_copy(g, g % 2, j).wait()

    return _sc_body


def _sc_part(x, hu_lis, norm_lis, B, Ma):
    nch = Ma // 16 // _CH
    assert Ma % (16 * _CH) == 0
    hu16 = jnp.pad(hu_lis, ((0, 0), (0, 8)))
    norm16 = jnp.pad(norm_lis, ((0, 0), (0, 8)))
    mesh = plsc.VectorSubcoreMesh(core_axis_name="c", subcore_axis_name="s")
    f = pl.kernel(
        _make_body(nch),
        out_type=jax.ShapeDtypeStruct((B * 4, Ma), jnp.float32),
        mesh=mesh,
        scratch_types=[
            pltpu.VMEM((4, 16), jnp.float32),
            pltpu.VMEM((4, 16), jnp.float32),
            pltpu.VMEM((5, 16), jnp.float32),
            pltpu.VMEM((5, 16), jnp.float32),
            pltpu.VMEM((5, 16), jnp.float32),
            pltpu.VMEM((16,), jnp.float32),
            pltpu.VMEM((2, _CH), jnp.float32),
            pltpu.VMEM((2, 4, _CH), jnp.float32),
            pltpu.SemaphoreType.DMA((2,)),
            pltpu.SemaphoreType.DMA((2,)),
        ],
        compiler_params=pltpu.CompilerParams(needs_layout_passes=False),
    )
    return f(x, hu16, norm16)


def _tc_body(hu_ref, norm_ref, x_ref, out_ref):
    x = x_ref[0, 0]
    for j in range(4):
        h_low = jnp.abs(hu_ref[j, 0])
        n_low = jnp.abs(norm_ref[j, 0])
        y = jnp.zeros_like(x)
        for i in range(1, 8):
            h_high = h_low + jnp.abs(hu_ref[j, i])
            n_high = n_low + jnp.abs(norm_ref[j, i])
            k = (n_high - n_low) / (h_high - h_low)
            c = n_low - k * h_low
            y = jnp.where(x >= _BASE_HU + h_low, k * x + c, y)
            h_low, n_low = h_high, n_high
        y = jnp.where(x >= _BASE_HU + h_low, n_low + _BASE_NORM, y)
        out_ref[0, j, 0] = y


def _merge_body(sc_ref, alias_ref, out_ref):
    del alias_ref
    out_ref[...] = sc_ref[...]


def kernel(img, hu_lis, norm_lis):
    B, C, D, H, W = img.shape
    M = D * H * W
    blk = _SUB * 256
    m = _SC_BLOCKS
    Ma = m * blk
    x = img.reshape(B, M)

    sc = _sc_part(x, hu_lis, norm_lis, B, Ma)

    x4 = x.reshape(B, 16, _SUB, 256)
    tc_full = pl.pallas_call(
        _tc_body,
        grid=(B, 16 - m),
        in_specs=[
            pl.BlockSpec(memory_space=pltpu.SMEM),
            pl.BlockSpec(memory_space=pltpu.SMEM),
            pl.BlockSpec((1, 1, _SUB, 256), lambda b, i: (b, i + m, 0, 0)),
        ],
        out_specs=pl.BlockSpec((1, 4, 1, _SUB, 256), lambda b, i: (b, 0, i + m, 0, 0)),
        out_shape=jax.ShapeDtypeStruct((B, 4, 16, _SUB, 256), jnp.float32),
    )(hu_lis, norm_lis, x4)

    sc5 = sc.reshape(B, 4, m, _SUB, 256)
    out = pl.pallas_call(
        _merge_body,
        grid=(B, m),
        in_specs=[
            pl.BlockSpec((1, 4, 1, _SUB, 256), lambda b, i: (b, 0, i, 0, 0)),
            pl.BlockSpec(memory_space=pl.ANY),
        ],
        out_specs=pl.BlockSpec((1, 4, 1, _SUB, 256), lambda b, i: (b, 0, i, 0, 0)),
        out_shape=jax.ShapeDtypeStruct((B, 4, 16, _SUB, 256), jnp.float32),
        input_output_aliases={1: 0},
    )(sc5, tc_full)

    return out.reshape(B, 4, D, H, W)


# speedup vs baseline: 3.1248x; 1.0973x over previous
import functools

import jax
import jax.numpy as jnp
from jax import lax
from jax.experimental import pallas as pl
from jax.experimental.pallas import tpu as pltpu
from jax.experimental.pallas import tpu_sc as plsc

_BASE_HU = -2.0
_BASE_NORM = 0.0

_NW = 32
_CH = 8192
_UNROLL = 4
_SUB = 512
_SC_BLOCKS = 3


def _cumsum16(x, tmp_v, iot):
    acc = x
    for s in (1, 2, 4, 8):
        tmp_v[...] = acc
        t = plsc.load_gather(tmp_v, [jnp.maximum(iot - s, 0)])
        acc = acc + jnp.where(iot >= s, t, 0.0)
    return acc


def _build_tables(hu_v, norm_v, bt_v, a_v, c_v, tmp_v):
    iot = lax.iota(jnp.int32, 16)
    inf = jnp.float32(jnp.inf)
    for j in range(4):
        habs = jnp.abs(hu_v[j])
        nabs = jnp.abs(norm_v[j])
        H = _cumsum16(habs, tmp_v, iot)
        N = _cumsum16(nabs, tmp_v, iot)
        Hprev = H - habs
        Nprev = N - nabs
        k = nabs / habs
        mid = (iot >= 1) & (iot <= 7)
        avec = jnp.where(mid, k, 0.0)
        cin = Nprev - k * Hprev
        cvec = jnp.where(iot == 8, Nprev + _BASE_NORM, jnp.where(mid, cin, 0.0))
        bvec = jnp.where(iot <= 7, _BASE_HU + H, inf)
        bt_v[j + 1] = bvec
        a_v[j + 1] = avec
        c_v[j + 1] = cvec


def _splats(bt_v, a_v, c_v, j):
    jv = jnp.full((16,), j + 1, jnp.int32)
    bs = [plsc.load_gather(bt_v, [jv, jnp.full((16,), i, jnp.int32)]) for i in range(8)]
    as_ = [plsc.load_gather(a_v, [jv, jnp.full((16,), i, jnp.int32)]) for i in range(1, 8)]
    cs = [plsc.load_gather(c_v, [jv, jnp.full((16,), i, jnp.int32)]) for i in range(1, 8)]
    top = plsc.load_gather(c_v, [jv, jnp.full((16,), 8, jnp.int32)])
    return bs, as_, cs, top


def _make_body(nch):
    def _sc_body(img_hbm, hu_hbm, norm_hbm, out_hbm,
                 hu_v, norm_v, bt_v, a_v, c_v, tmp_v, x_v, y_v,
                 in_sems, out_sems):
        c = lax.axis_index("c")
        s = lax.axis_index("s")
        wid = s * 2 + c
        b = wid // 16
        col0 = (wid % 16) * (_CH * nch)

        pltpu.sync_copy(hu_hbm, hu_v)
        pltpu.sync_copy(norm_hbm, norm_v)
        _build_tables(hu_v, norm_v, bt_v, a_v, c_v, tmp_v)

        def in_copy(g, buf):
            return pltpu.make_async_copy(
                img_hbm.at[b, pl.ds(col0 + g * _CH, _CH)], x_v.at[buf], in_sems.at[buf])

        def out_copy(g, buf, j):
            col = col0 + g * _CH
            bi = col // (_SUB * 256)
            r0 = pl.multiple_of((col % (_SUB * 256)) // 256, _CH // 256)
            return pltpu.make_async_copy(
                y_v.at[buf, j],
                out_hbm.at[b, j, bi, pl.ds(r0, _CH // 256)],
                out_sems.at[buf])

        in_copy(0, 0).start()

        def chunk_body(g, _):
            buf = lax.rem(g, 2)
            in_copy(g, buf).wait()

            @pl.when(g + 1 < nch)
            def _():
                in_copy(g + 1, lax.rem(g + 1, 2)).start()

            @pl.when(g >= 2)
            def _():
                for j in range(4):
                    out_copy(g - 2, buf, j).wait()

            for j in range(4):
                bs, as_, cs, top = _splats(bt_v, a_v, c_v, j)

                def vec_body(i, _, buf=buf, j=j, bs=bs, as_=as_, cs=cs, top=top):
                    for u in range(_UNROLL):
                        off = (i * _UNROLL + u) * 16
                        x = x_v[buf, pl.ds(off, 16)]
                        y = jnp.zeros((16,), jnp.float32)
                        for t in range(7):
                            y = jnp.where(x >= bs[t], as_[t] * x + cs[t], y)
                        y = jnp.where(x >= bs[7], top, y)
                        y_v[buf, j, off // 256, pl.ds(off % 256, 16)] = y
                    return 0

                lax.fori_loop(0, _CH // (16 * _UNROLL), vec_body, 0)
                out_copy(g, buf, j).start()
            return 0

        lax.fori_loop(0, nch, chunk_body, 0)
        for g in (nch - 2, nch - 1):
            for j in range(4):
                out_copy(g, g % 2, j).wait()

    return _sc_body


def _sc_part(x, hu_lis, norm_lis, B, Ma):
    nch = Ma // 16 // _CH
    assert Ma % (16 * _CH) == 0
    hu16 = jnp.pad(hu_lis, ((0, 0), (0, 8)))
    norm16 = jnp.pad(norm_lis, ((0, 0), (0, 8)))
    mesh = plsc.VectorSubcoreMesh(core_axis_name="c", subcore_axis_name="s")
    f = pl.kernel(
        _make_body(nch),
        out_type=jax.ShapeDtypeStruct((B, 4, Ma // (_SUB * 256), _SUB, 256), jnp.float32),
        mesh=mesh,
        scratch_types=[
            pltpu.VMEM((4, 16), jnp.float32),
            pltpu.VMEM((4, 16), jnp.float32),
            pltpu.VMEM((5, 16), jnp.float32),
            pltpu.VMEM((5, 16), jnp.float32),
            pltpu.VMEM((5, 16), jnp.float32),
            pltpu.VMEM((16,), jnp.float32),
            pltpu.VMEM((2, _CH), jnp.float32),
            pltpu.VMEM((2, 4, _CH // 256, 256), jnp.float32),
            pltpu.SemaphoreType.DMA((2,)),
            pltpu.SemaphoreType.DMA((2,)),
        ],
        compiler_params=pltpu.CompilerParams(needs_layout_passes=False),
    )
    return f(x, hu16, norm16)


def _tc_body(hu_ref, norm_ref, x_ref, out_ref):
    x = x_ref[0, 0]
    for j in range(4):
        h_low = jnp.abs(hu_ref[j, 0])
        n_low = jnp.abs(norm_ref[j, 0])
        y = jnp.zeros_like(x)
        for i in range(1, 8):
            h_high = h_low + jnp.abs(hu_ref[j, i])
            n_high = n_low + jnp.abs(norm_ref[j, i])
            k = (n_high - n_low) / (h_high - h_low)
            c = n_low - k * h_low
            y = jnp.where(x >= _BASE_HU + h_low, k * x + c, y)
            h_low, n_low = h_high, n_high
        y = jnp.where(x >= _BASE_HU + h_low, n_low + _BASE_NORM, y)
        out_ref[0, j, 0] = y


def _merge_body(sc_ref, alias_ref, out_ref):
    del alias_ref
    out_ref[...] = sc_ref[...]


def kernel(img, hu_lis, norm_lis):
    B, C, D, H, W = img.shape
    M = D * H * W
    blk = _SUB * 256
    m = _SC_BLOCKS
    Ma = m * blk
    x = img.reshape(B, M)

    sc = _sc_part(x, hu_lis, norm_lis, B, Ma)

    x4 = x.reshape(B, 16, _SUB, 256)
    tc_full = pl.pallas_call(
        _tc_body,
        grid=(B, 16 - m),
        in_specs=[
            pl.BlockSpec(memory_space=pltpu.SMEM),
            pl.BlockSpec(memory_space=pltpu.SMEM),
            pl.BlockSpec((1, 1, _SUB, 256), lambda b, i: (b, i + m, 0, 0)),
        ],
        out_specs=pl.BlockSpec((1, 4, 1, _SUB, 256), lambda b, i: (b, 0, i + m, 0, 0)),
        out_shape=jax.ShapeDtypeStruct((B, 4, 16, _SUB, 256), jnp.float32),
    )(hu_lis, norm_lis, x4)

    out = pl.pallas_call(
        _merge_body,
        grid=(B, m),
        in_specs=[
            pl.BlockSpec((1, 4, 1, _SUB, 256), lambda b, i: (b, 0, i, 0, 0)),
            pl.BlockSpec(memory_space=pl.ANY),
        ],
        out_specs=pl.BlockSpec((1, 4, 1, _SUB, 256), lambda b, i: (b, 0, i, 0, 0)),
        out_shape=jax.ShapeDtypeStruct((B, 4, 16, _SUB, 256), jnp.float32),
        input_output_aliases={1: 0},
    )(sc, tc_full)

    return out.reshape(B, 4, D, H, W)
